# baseline (speedup 1.0000x reference)
import functools
import jax
import jax.numpy as jnp
from jax import lax
from jax.experimental import pallas as pl
from jax.experimental.pallas import tpu as pltpu

_ALPHA = 1000.0
_BETA = 100.0
_EMBED = 72
_K = 64
_STAGES = 2
_INTERPRET = False


def _pose_initial_body(x_ref, rgbx_ref, o_ref):
    B, C, N = o_ref.shape
    cidx = lax.broadcasted_iota(jnp.int32, (B, C, N), 1)
    q = cidx % 24
    p = (q // 2).astype(jnp.float32)
    is_sin = (q % 2) == 0
    j = cidx // 24
    coef = _BETA * jnp.exp(-(p / 12.0) * jnp.log(jnp.float32(_ALPHA)))

    def emb(t):
        t0 = t[:, 0, :][:, None, :]
        t1 = t[:, 1, :][:, None, :]
        t2 = t[:, 2, :][:, None, :]
        tf = jnp.where(j == 0, t0, jnp.where(j == 1, t1, t2))
        m = coef * tf
        return jnp.where(is_sin, jnp.sin(m), jnp.cos(m))

    o_ref[...] = 0.8 * emb(x_ref[...]) + 0.2 * emb(rgbx_ref[...])


def _pose_initial(x, rgbx):
    B, _, N = x.shape
    return pl.pallas_call(
        _pose_initial_body,
        out_shape=jax.ShapeDtypeStruct((B, _EMBED, N), jnp.float32),
        interpret=_INTERPRET,
    )(x, rgbx)


def _index_points(points, idx):
    B = points.shape[0]
    bidx = jnp.arange(B).reshape((B,) + (1,) * (idx.ndim - 1))
    return points[bidx, idx]


def _fps_body(npoint, xyzt_ref, idx_ref, dist_ref):
    B, _, N = xyzt_ref.shape
    dist_ref[...] = jnp.full((B, N), 1e10, dtype=jnp.float32)
    lane = lax.broadcasted_iota(jnp.int32, (B, N), 1)
    col = lax.broadcasted_iota(jnp.int32, (B, npoint), 1)

    idx_ref[...] = jnp.zeros((B, npoint), jnp.int32)

    def step(i, far):
        idx_ref[...] = jnp.where(col == i, far, idx_ref[...])
        oh = (lane == far).astype(jnp.float32)
        d = jnp.zeros((B, N), jnp.float32)
        for j in range(3):
            row = xyzt_ref[:, j, :]
            cj = jnp.sum(row * oh, axis=1, keepdims=True)
            d = d + (row - cj) ** 2
        nd = jnp.minimum(dist_ref[...], d)
        dist_ref[...] = nd
        m = jnp.max(nd, axis=1, keepdims=True)
        return jnp.min(jnp.where(nd == m, lane, N), axis=1, keepdims=True)

    lax.fori_loop(0, npoint, step, jnp.zeros((B, 1), jnp.int32))


def _fps(xyz, npoint):
    B, N, _ = xyz.shape
    xyzt = jnp.transpose(xyz, (0, 2, 1))
    return pl.pallas_call(
        functools.partial(_fps_body, npoint),
        out_shape=jax.ShapeDtypeStruct((B, npoint), jnp.int32),
        scratch_shapes=[pltpu.VMEM((B, N), jnp.float32)],
        interpret=_INTERPRET,
    )(xyzt)


def _knn_body(k, xyzt_ref, lct_ref, idx_ref):
    _, _, N = xyzt_ref.shape
    _, _, BG = lct_ref.shape
    qsq = jnp.zeros((BG, 1), jnp.float32)
    psq = jnp.zeros((1, N), jnp.float32)
    dot = jnp.zeros((BG, N), jnp.float32)
    for j in range(3):
        q = lct_ref[0, j, :][:, None]
        p = xyzt_ref[0, j, :][None, :]
        qsq = qsq + q * q
        psq = psq + p * p
        q16 = q.astype(jnp.bfloat16).astype(jnp.float32)
        p16 = p.astype(jnp.bfloat16).astype(jnp.float32)
        dot = dot + q16 * p16
    d = qsq + psq - 2.0 * dot
    lane = lax.broadcasted_iota(jnp.int32, (BG, N), 1)
    col = lax.broadcasted_iota(jnp.int32, (BG, k), 1)
    acc = jnp.zeros((BG, k), jnp.int32)
    for kk in range(k):
        m = jnp.min(d, axis=1, keepdims=True)
        ai = jnp.min(jnp.where(d == m, lane, N), axis=1, keepdims=True)
        acc = jnp.where(col == kk, ai, acc)
        d = jnp.where(lane == ai, jnp.float32(jnp.inf), d)
    idx_ref[0] = acc


def _knn_xla(k, xyz, new_xyz):
    sq = (jnp.sum(new_xyz ** 2, -1)[..., None]
          + jnp.sum(xyz ** 2, -1)[:, None, :]
          - 2.0 * jnp.einsum('bgd,bnd->bgn', new_xyz, xyz))
    _, idx = jax.lax.top_k(-sq, k)
    return idx


def _knn(k, xyz, lc_xyz):
    B, N, _ = xyz.shape
    G = lc_xyz.shape[1]
    BG = 256
    xyzt = jnp.transpose(xyz, (0, 2, 1))
    lct = jnp.transpose(lc_xyz, (0, 2, 1))
    return pl.pallas_call(
        functools.partial(_knn_body, k),
        grid=(B, G // BG),
        in_specs=[
            pl.BlockSpec((1, 3, N), lambda b, g: (b, 0, 0)),
            pl.BlockSpec((1, 3, BG), lambda b, g: (b, 0, g)),
        ],
        out_specs=pl.BlockSpec((1, BG, k), lambda b, g: (b, g, 0)),
        out_shape=jax.ShapeDtypeStruct((B, G, k), jnp.int32),
        interpret=_INTERPRET,
    )(xyzt, lct)


def _sel3(t, j_idx):
    t0 = t[:, 0:1]
    t1 = t[:, 1:2]
    t2 = t[:, 2:3]
    return jnp.where(j_idx == 0, t0, jnp.where(j_idx == 1, t1, t2))


def _extract(v, col, i):
    return jnp.sum(v * (col == i).astype(jnp.float32))


def _lga_stats_body(knn_x_ref, lc_x_ref, knn_xyz_ref, lc_xyz_ref, out_ref):
    b = pl.program_id(0)
    g = pl.program_id(1)
    _, GB, K, C = knn_x_ref.shape
    dx = knn_x_ref[0] - lc_x_ref[0][:, None, :]
    dz = knn_xyz_ref[0] - lc_xyz_ref[0][:, None, :]
    col = lax.broadcasted_iota(jnp.int32, (1, 4), 1)
    row = (jnp.sum(dx) * (col == 0) + jnp.sum(dx * dx) * (col == 1)
           + jnp.sum(dz) * (col == 2) + jnp.sum(dz * dz) * (col == 3))

    @pl.when(jnp.logical_and(b == 0, g == 0))
    def _():
        out_ref[...] = jnp.zeros_like(out_ref)

    out_ref[...] += row


def _lga_main_body(nx, nz, knn_x_ref, lc_x_ref, knn_xyz_ref, lc_xyz_ref,
                   knn_rgb_ref, vvt_ref, stats_ref, pooled_ref, acc_ref):
    b = pl.program_id(0)
    g = pl.program_id(1)
    _, GB, K, C = knn_x_ref.shape
    O = 2 * C
    fd = O // 6
    R = GB * K

    sv = stats_ref[...]
    scol = lax.broadcasted_iota(jnp.int32, (1, 4), 1)
    sdx = _extract(sv, scol, 0)
    sdx2 = _extract(sv, scol, 1)
    sdz = _extract(sv, scol, 2)
    sdz2 = _extract(sv, scol, 3)
    std_x = jnp.sqrt((sdx2 - sdx * sdx / nx) / (nx - 1.0))
    std_z = jnp.sqrt((sdz2 - sdz * sdz / nz) / (nz - 1.0))
    mx_inv = 1.0 / (std_x + 1e-05)
    mz_inv = 1.0 / (std_z + 1e-05)

    lc_x = lc_x_ref[0]
    knn_xn = ((knn_x_ref[0] - lc_x[:, None, :]) * mx_inv).reshape(R, C)
    knn_zn = ((knn_xyz_ref[0] - lc_xyz_ref[0][:, None, :]) * mz_inv).reshape(R, 3)
    rgb = knn_rgb_ref[0].reshape(R, 3)
    lc_b = jnp.broadcast_to(lc_x[:, None, :], (GB, K, C)).reshape(R, C)

    cidx = lax.broadcasted_iota(jnp.int32, (R, O), 1)
    j_idx = cidx // (2 * fd)
    is_sin = (cidx % (2 * fd)) < fd
    z_sel = _sel3(knn_zn, j_idx)
    r_sel = _sel3(rgb, j_idx)
    ez = jnp.where(is_sin, jnp.sin(z_sel), jnp.cos(z_sel))
    er = jnp.where(is_sin, jnp.sin(r_sel), jnp.cos(r_sel))
    cat = jnp.concatenate([knn_xn, lc_b], axis=1)
    mix = cat / 3.0 + ez / 3.0 + er / 3.0

    ocol = lax.broadcasted_iota(jnp.int32, (O, O), 1).astype(jnp.float32)
    W = jnp.cos(vvt_ref[...] * ocol * (2.0 * jnp.pi))
    y = jnp.dot(mix.astype(jnp.bfloat16), W.astype(jnp.bfloat16),
                preferred_element_type=jnp.float32)

    pooled_ref[0] = jnp.max(y.reshape(GB, K, O), axis=1)
    acol = lax.broadcasted_iota(jnp.int32, (1, 4), 1)
    row = jnp.sum(y) * (acol == 0) + jnp.sum(y * y) * (acol == 1)

    @pl.when(jnp.logical_and(b == 0, g == 0))
    def _():
        acc_ref[...] = jnp.zeros_like(acc_ref)

    acc_ref[...] += row


def _lga_final_body(n, pooled_ref, acc_ref, out_ref):
    sv = acc_ref[...]
    scol = lax.broadcasted_iota(jnp.int32, (1, 4), 1)
    sy = _extract(sv, scol, 0)
    sy2 = _extract(sv, scol, 1)
    m = sy / n
    s = jnp.sqrt((sy2 - sy * sy / n) / (n - 1.0))
    z = (pooled_ref[...] - m) / (s + 1e-06)
    out_ref[...] = jnp.where(z > 0, z, 0.1 * z)


def _lga_fused(lc_xyz, lc_x, lc_rgb, knn_xyz, knn_x, knn_rgb, out_dim, vv):
    B, G, K, C = knn_x.shape
    O = 2 * C
    GB = 128 if C <= 72 else 64
    nx = float(B * G * K * C)
    nz = float(B * G * K * 3)
    ny = float(B * G * K * O)
    vvt = jnp.reshape(vv, (vv.shape[1], 1))

    stats = pl.pallas_call(
        _lga_stats_body,
        grid=(B, G // GB),
        in_specs=[
            pl.BlockSpec((1, GB, K, C), lambda b, g: (b, g, 0, 0)),
            pl.BlockSpec((1, GB, C), lambda b, g: (b, g, 0)),
            pl.BlockSpec((1, GB, K, 3), lambda b, g: (b, g, 0, 0)),
            pl.BlockSpec((1, GB, 3), lambda b, g: (b, g, 0)),
        ],
        out_specs=pl.BlockSpec((1, 4), lambda b, g: (0, 0)),
        out_shape=jax.ShapeDtypeStruct((1, 4), jnp.float32),
        interpret=_INTERPRET,
    )(knn_x, lc_x, knn_xyz, lc_xyz)

    pooled, acc = pl.pallas_call(
        functools.partial(_lga_main_body, nx, nz),
        grid=(B, G // GB),
        in_specs=[
            pl.BlockSpec((1, GB, K, C), lambda b, g: (b, g, 0, 0)),
            pl.BlockSpec((1, GB, C), lambda b, g: (b, g, 0)),
            pl.BlockSpec((1, GB, K, 3), lambda b, g: (b, g, 0, 0)),
            pl.BlockSpec((1, GB, 3), lambda b, g: (b, g, 0)),
            pl.BlockSpec((1, GB, K, 3), lambda b, g: (b, g, 0, 0)),
            pl.BlockSpec((O, 1), lambda b, g: (0, 0)),
            pl.BlockSpec((1, 4), lambda b, g: (0, 0)),
        ],
        out_specs=[
            pl.BlockSpec((1, GB, O), lambda b, g: (b, g, 0)),
            pl.BlockSpec((1, 4), lambda b, g: (0, 0)),
        ],
        out_shape=[
            jax.ShapeDtypeStruct((B, G, O), jnp.float32),
            jax.ShapeDtypeStruct((1, 4), jnp.float32),
        ],
        interpret=_INTERPRET,
    )(knn_x, lc_x, knn_xyz, lc_xyz, knn_rgb, vvt, stats)

    x_new = pl.pallas_call(
        functools.partial(_lga_final_body, ny),
        interpret=_INTERPRET,
        out_shape=jax.ShapeDtypeStruct((B, G, O), jnp.float32),
    )(pooled, acc)
    return x_new


def _pose_geo(knn_xyz, knn_x, knn_rgb, out_dim, vv):
    B, _, G, K = knn_xyz.shape
    feat_dim = out_dim // 6

    def embed(t):
        t1 = jnp.transpose(t, (0, 2, 3, 1))[..., None]
        div = jnp.broadcast_to(t1, t1.shape[:-1] + (feat_dim,))
        e = jnp.concatenate([jnp.sin(div), jnp.cos(div)], axis=4)
        e = e.reshape(B, G, K, out_dim)
        return jnp.transpose(e, (0, 3, 1, 2))

    xyz_embed = embed(knn_xyz)
    rgb_embed = embed(knn_rgb)
    pos = vv[:, :out_dim].T @ jnp.arange(out_dim, dtype=jnp.float32)[None, :]
    W_l = jnp.cos(pos * 2.0 * jnp.pi)
    knn_x_new = knn_x / 3.0 + xyz_embed / 3.0 + rgb_embed / 3.0
    knn_x_new = jnp.transpose(knn_x_new, (0, 2, 3, 1))
    knn_x_new = knn_x_new @ W_l
    m = jnp.mean(knn_x_new)
    s = jnp.std(knn_x_new - m, ddof=1)
    knn_x_new = (knn_x_new - m) / (s + 1e-06)
    return jnp.transpose(knn_x_new, (0, 3, 1, 2))


def _lga(lc_xyz, lc_x, lc_rgb, knn_xyz, knn_x, knn_rgb, out_dim, vv):
    mean_x = lc_x[:, :, None, :]
    std_x = jnp.std(knn_x - mean_x, ddof=1)
    mean_xyz = lc_xyz[:, :, None, :]
    std_xyz = jnp.std(knn_xyz - mean_xyz, ddof=1)
    knn_x = (knn_x - mean_x) / (std_x + 1e-05)
    knn_xyz = (knn_xyz - mean_xyz) / (std_xyz + 1e-05)
    B, G, K, C = knn_x.shape
    knn_x = jnp.concatenate(
        [knn_x, jnp.broadcast_to(lc_x[:, :, None, :], (B, G, K, C))], axis=-1)
    return _pose_geo(jnp.transpose(knn_xyz, (0, 3, 1, 2)),
                     jnp.transpose(knn_x, (0, 3, 1, 2)),
                     jnp.transpose(knn_rgb, (0, 3, 1, 2)), out_dim, vv)


def kernel(xyz, x, rgb, rgbx, vv):
    x = _pose_initial(x, rgbx)
    xyz_list = [xyz]
    x_list = [x]
    out_dim = _EMBED
    group_num = xyz.shape[1]
    for i in range(_STAGES):
        out_dim = out_dim * 2
        group_num = group_num // 2
        x_t = jnp.transpose(x, (0, 2, 1))
        fps_idx = _fps(xyz, group_num)
        lc_xyz = _index_points(xyz, fps_idx)
        lc_x = _index_points(x_t, fps_idx)
        lc_rgb = _index_points(rgb, fps_idx)
        knn_idx = _knn(_K, xyz, lc_xyz)
        knn_xyz = _index_points(xyz, knn_idx)
        knn_x = _index_points(x_t, knn_idx)
        knn_rgb = _index_points(rgb, knn_idx)
        x_new = _lga_fused(lc_xyz, lc_x, lc_rgb, knn_xyz, knn_x, knn_rgb,
                           out_dim, vv)
        x = jnp.transpose(x_new, (0, 2, 1))
        xyz = lc_xyz
        rgb = lc_rgb
        xyz_list.append(xyz)
        x_list.append(x)
    return (tuple(xyz_list), tuple(x_list))


# speedup vs baseline: 1.2465x; 1.2465x over previous
import functools
import jax
import jax.numpy as jnp
from jax import lax
from jax.experimental import pallas as pl
from jax.experimental.pallas import tpu as pltpu

_ALPHA = 1000.0
_BETA = 100.0
_EMBED = 72
_K = 64
_STAGES = 2
_INTERPRET = False


def _pose_initial_body(x_ref, rgbx_ref, o_ref):
    B, C, N = o_ref.shape
    cidx = lax.broadcasted_iota(jnp.int32, (B, C, N), 1)
    q = cidx % 24
    p = (q // 2).astype(jnp.float32)
    is_sin = (q % 2) == 0
    j = cidx // 24
    coef = _BETA * jnp.exp(-(p / 12.0) * jnp.log(jnp.float32(_ALPHA)))

    def emb(t):
        t0 = t[:, 0, :][:, None, :]
        t1 = t[:, 1, :][:, None, :]
        t2 = t[:, 2, :][:, None, :]
        tf = jnp.where(j == 0, t0, jnp.where(j == 1, t1, t2))
        m = coef * tf
        return jnp.where(is_sin, jnp.sin(m), jnp.cos(m))

    o_ref[...] = 0.8 * emb(x_ref[...]) + 0.2 * emb(rgbx_ref[...])


def _pose_initial(x, rgbx):
    B, _, N = x.shape
    return pl.pallas_call(
        _pose_initial_body,
        out_shape=jax.ShapeDtypeStruct((B, _EMBED, N), jnp.float32),
        interpret=_INTERPRET,
    )(x, rgbx)


def _index_points(points, idx):
    B = points.shape[0]
    bidx = jnp.arange(B).reshape((B,) + (1,) * (idx.ndim - 1))
    return points[bidx, idx]


def _fps_body(npoint, xyzt_ref, idx_ref, dist_ref):
    B, _, N = xyzt_ref.shape
    dist_ref[...] = jnp.full((B, N), 1e10, dtype=jnp.float32)
    lane = lax.broadcasted_iota(jnp.int32, (B, N), 1)
    col = lax.broadcasted_iota(jnp.int32, (B, npoint), 1)

    idx_ref[...] = jnp.zeros((B, npoint), jnp.int32)

    def step(i, far):
        idx_ref[...] = jnp.where(col == i, far, idx_ref[...])
        oh = (lane == far).astype(jnp.float32)
        d = jnp.zeros((B, N), jnp.float32)
        for j in range(3):
            row = xyzt_ref[:, j, :]
            cj = jnp.sum(row * oh, axis=1, keepdims=True)
            d = d + (row - cj) ** 2
        nd = jnp.minimum(dist_ref[...], d)
        dist_ref[...] = nd
        m = jnp.max(nd, axis=1, keepdims=True)
        return jnp.min(jnp.where(nd == m, lane, N), axis=1, keepdims=True)

    lax.fori_loop(0, npoint, step, jnp.zeros((B, 1), jnp.int32))


def _fps(xyz, npoint):
    B, N, _ = xyz.shape
    xyzt = jnp.transpose(xyz, (0, 2, 1))
    return pl.pallas_call(
        functools.partial(_fps_body, npoint),
        out_shape=jax.ShapeDtypeStruct((B, npoint), jnp.int32),
        scratch_shapes=[pltpu.VMEM((B, N), jnp.float32)],
        interpret=_INTERPRET,
    )(xyzt)


def _knn_body(k, xyzt_ref, lct_ref, idx_ref):
    _, _, N = xyzt_ref.shape
    _, _, BG = lct_ref.shape
    qsq = jnp.zeros((BG, 1), jnp.float32)
    psq = jnp.zeros((1, N), jnp.float32)
    dot = jnp.zeros((BG, N), jnp.float32)
    for j in range(3):
        q = lct_ref[0, j, :][:, None]
        p = xyzt_ref[0, j, :][None, :]
        qsq = qsq + q * q
        psq = psq + p * p
        q16 = q.astype(jnp.bfloat16).astype(jnp.float32)
        p16 = p.astype(jnp.bfloat16).astype(jnp.float32)
        dot = dot + q16 * p16
    d = qsq + psq - 2.0 * dot
    lane = lax.broadcasted_iota(jnp.int32, (BG, N), 1)
    col = lax.broadcasted_iota(jnp.int32, (BG, k), 1)
    acc = jnp.zeros((BG, k), jnp.int32)
    for kk in range(k):
        m = jnp.min(d, axis=1, keepdims=True)
        ai = jnp.min(jnp.where(d == m, lane, N), axis=1, keepdims=True)
        acc = jnp.where(col == kk, ai, acc)
        d = jnp.where(lane == ai, jnp.float32(jnp.inf), d)
    idx_ref[0] = acc


def _knn_xla(k, xyz, new_xyz):
    sq = (jnp.sum(new_xyz ** 2, -1)[..., None]
          + jnp.sum(xyz ** 2, -1)[:, None, :]
          - 2.0 * jnp.einsum('bgd,bnd->bgn', new_xyz, xyz))
    _, idx = jax.lax.top_k(-sq, k)
    return idx


def _knn(k, xyz, lc_xyz):
    B, N, _ = xyz.shape
    G = lc_xyz.shape[1]
    BG = 256
    xyzt = jnp.transpose(xyz, (0, 2, 1))
    lct = jnp.transpose(lc_xyz, (0, 2, 1))
    return pl.pallas_call(
        functools.partial(_knn_body, k),
        grid=(B, G // BG),
        in_specs=[
            pl.BlockSpec((1, 3, N), lambda b, g: (b, 0, 0)),
            pl.BlockSpec((1, 3, BG), lambda b, g: (b, 0, g)),
        ],
        out_specs=pl.BlockSpec((1, BG, k), lambda b, g: (b, g, 0)),
        out_shape=jax.ShapeDtypeStruct((B, G, k), jnp.int32),
        interpret=_INTERPRET,
    )(xyzt, lct)


def _sel3(t, j_idx):
    t0 = t[:, 0:1]
    t1 = t[:, 1:2]
    t2 = t[:, 2:3]
    return jnp.where(j_idx == 0, t0, jnp.where(j_idx == 1, t1, t2))


def _extract(v, col, i):
    return jnp.sum(v * (col == i).astype(jnp.float32))


def _lga_stats_body(knn_x_ref, lc_x_ref, knn_xyz_ref, lc_xyz_ref, out_ref):
    b = pl.program_id(0)
    g = pl.program_id(1)
    _, GB, K, C = knn_x_ref.shape
    dx = knn_x_ref[0] - lc_x_ref[0][:, None, :]
    dz = knn_xyz_ref[0] - lc_xyz_ref[0][:, None, :]
    col = lax.broadcasted_iota(jnp.int32, (1, 4), 1)
    row = (jnp.sum(dx) * (col == 0) + jnp.sum(dx * dx) * (col == 1)
           + jnp.sum(dz) * (col == 2) + jnp.sum(dz * dz) * (col == 3))

    @pl.when(jnp.logical_and(b == 0, g == 0))
    def _():
        out_ref[...] = jnp.zeros_like(out_ref)

    out_ref[...] += row


def _lga_main_body(nx, nz, knn_x_ref, lc_x_ref, knn_xyz_ref, lc_xyz_ref,
                   knn_rgb_ref, vvt_ref, stats_ref, pooled_ref, acc_ref):
    b = pl.program_id(0)
    g = pl.program_id(1)
    _, GB, K, C = knn_x_ref.shape
    O = 2 * C
    fd = O // 6
    R = GB * K

    sv = stats_ref[...]
    scol = lax.broadcasted_iota(jnp.int32, (1, 4), 1)
    sdx = _extract(sv, scol, 0)
    sdx2 = _extract(sv, scol, 1)
    sdz = _extract(sv, scol, 2)
    sdz2 = _extract(sv, scol, 3)
    std_x = jnp.sqrt((sdx2 - sdx * sdx / nx) / (nx - 1.0))
    std_z = jnp.sqrt((sdz2 - sdz * sdz / nz) / (nz - 1.0))
    mx_inv = 1.0 / (std_x + 1e-05)
    mz_inv = 1.0 / (std_z + 1e-05)

    lc_x = lc_x_ref[0]
    knn_xn = ((knn_x_ref[0] - lc_x[:, None, :]) * mx_inv).reshape(R, C)
    knn_zn = ((knn_xyz_ref[0] - lc_xyz_ref[0][:, None, :]) * mz_inv).reshape(R, 3)
    rgb = knn_rgb_ref[0].reshape(R, 3)
    lc_b = jnp.broadcast_to(lc_x[:, None, :], (GB, K, C)).reshape(R, C)

    cidx = lax.broadcasted_iota(jnp.int32, (R, O), 1)
    j_idx = cidx // (2 * fd)
    is_sin = (cidx % (2 * fd)) < fd
    z_sel = _sel3(knn_zn, j_idx)
    r_sel = _sel3(rgb, j_idx)
    ez = jnp.where(is_sin, jnp.sin(z_sel), jnp.cos(z_sel))
    er = jnp.where(is_sin, jnp.sin(r_sel), jnp.cos(r_sel))
    cat = jnp.concatenate([knn_xn, lc_b], axis=1)
    mix = cat / 3.0 + ez / 3.0 + er / 3.0

    ocol = lax.broadcasted_iota(jnp.int32, (O, O), 1).astype(jnp.float32)
    W = jnp.cos(vvt_ref[...] * ocol * (2.0 * jnp.pi))
    y = jnp.dot(mix.astype(jnp.bfloat16), W.astype(jnp.bfloat16),
                preferred_element_type=jnp.float32)

    pooled_ref[0] = jnp.max(y.reshape(GB, K, O), axis=1)
    acol = lax.broadcasted_iota(jnp.int32, (1, 4), 1)
    row = jnp.sum(y) * (acol == 0) + jnp.sum(y * y) * (acol == 1)

    @pl.when(jnp.logical_and(b == 0, g == 0))
    def _():
        acc_ref[...] = jnp.zeros_like(acc_ref)

    acc_ref[...] += row


def _lga_final_body(n, pooled_ref, acc_ref, out_ref):
    sv = acc_ref[...]
    scol = lax.broadcasted_iota(jnp.int32, (1, 4), 1)
    sy = _extract(sv, scol, 0)
    sy2 = _extract(sv, scol, 1)
    m = sy / n
    s = jnp.sqrt((sy2 - sy * sy / n) / (n - 1.0))
    z = (pooled_ref[...] - m) / (s + 1e-06)
    out_ref[...] = jnp.where(z > 0, z, 0.1 * z)


def _lga_fused(lc_xyz, lc_x, lc_rgb, knn_xyz, knn_x, knn_rgb, out_dim, vv):
    B, G, K, C = knn_x.shape
    O = 2 * C
    GB = 128 if C <= 72 else 64
    nx = float(B * G * K * C)
    nz = float(B * G * K * 3)
    ny = float(B * G * K * O)
    vvt = jnp.reshape(vv, (vv.shape[1], 1))

    stats = pl.pallas_call(
        _lga_stats_body,
        grid=(B, G // GB),
        in_specs=[
            pl.BlockSpec((1, GB, K, C), lambda b, g: (b, g, 0, 0)),
            pl.BlockSpec((1, GB, C), lambda b, g: (b, g, 0)),
            pl.BlockSpec((1, GB, K, 3), lambda b, g: (b, g, 0, 0)),
            pl.BlockSpec((1, GB, 3), lambda b, g: (b, g, 0)),
        ],
        out_specs=pl.BlockSpec((1, 4), lambda b, g: (0, 0)),
        out_shape=jax.ShapeDtypeStruct((1, 4), jnp.float32),
        interpret=_INTERPRET,
    )(knn_x, lc_x, knn_xyz, lc_xyz)

    pooled, acc = pl.pallas_call(
        functools.partial(_lga_main_body, nx, nz),
        grid=(B, G // GB),
        in_specs=[
            pl.BlockSpec((1, GB, K, C), lambda b, g: (b, g, 0, 0)),
            pl.BlockSpec((1, GB, C), lambda b, g: (b, g, 0)),
            pl.BlockSpec((1, GB, K, 3), lambda b, g: (b, g, 0, 0)),
            pl.BlockSpec((1, GB, 3), lambda b, g: (b, g, 0)),
            pl.BlockSpec((1, GB, K, 3), lambda b, g: (b, g, 0, 0)),
            pl.BlockSpec((O, 1), lambda b, g: (0, 0)),
            pl.BlockSpec((1, 4), lambda b, g: (0, 0)),
        ],
        out_specs=[
            pl.BlockSpec((1, GB, O), lambda b, g: (b, g, 0)),
            pl.BlockSpec((1, 4), lambda b, g: (0, 0)),
        ],
        out_shape=[
            jax.ShapeDtypeStruct((B, G, O), jnp.float32),
            jax.ShapeDtypeStruct((1, 4), jnp.float32),
        ],
        interpret=_INTERPRET,
    )(knn_x, lc_x, knn_xyz, lc_xyz, knn_rgb, vvt, stats)

    x_new = pl.pallas_call(
        functools.partial(_lga_final_body, ny),
        interpret=_INTERPRET,
        out_shape=jax.ShapeDtypeStruct((B, G, O), jnp.float32),
    )(pooled, acc)
    return x_new


def _pose_geo(knn_xyz, knn_x, knn_rgb, out_dim, vv):
    B, _, G, K = knn_xyz.shape
    feat_dim = out_dim // 6

    def embed(t):
        t1 = jnp.transpose(t, (0, 2, 3, 1))[..., None]
        div = jnp.broadcast_to(t1, t1.shape[:-1] + (feat_dim,))
        e = jnp.concatenate([jnp.sin(div), jnp.cos(div)], axis=4)
        e = e.reshape(B, G, K, out_dim)
        return jnp.transpose(e, (0, 3, 1, 2))

    xyz_embed = embed(knn_xyz)
    rgb_embed = embed(knn_rgb)
    pos = vv[:, :out_dim].T @ jnp.arange(out_dim, dtype=jnp.float32)[None, :]
    W_l = jnp.cos(pos * 2.0 * jnp.pi)
    knn_x_new = knn_x / 3.0 + xyz_embed / 3.0 + rgb_embed / 3.0
    knn_x_new = jnp.transpose(knn_x_new, (0, 2, 3, 1))
    knn_x_new = knn_x_new @ W_l
    m = jnp.mean(knn_x_new)
    s = jnp.std(knn_x_new - m, ddof=1)
    knn_x_new = (knn_x_new - m) / (s + 1e-06)
    return jnp.transpose(knn_x_new, (0, 3, 1, 2))


def _lga(lc_xyz, lc_x, lc_rgb, knn_xyz, knn_x, knn_rgb, out_dim, vv):
    mean_x = lc_x[:, :, None, :]
    std_x = jnp.std(knn_x - mean_x, ddof=1)
    mean_xyz = lc_xyz[:, :, None, :]
    std_xyz = jnp.std(knn_xyz - mean_xyz, ddof=1)
    knn_x = (knn_x - mean_x) / (std_x + 1e-05)
    knn_xyz = (knn_xyz - mean_xyz) / (std_xyz + 1e-05)
    B, G, K, C = knn_x.shape
    knn_x = jnp.concatenate(
        [knn_x, jnp.broadcast_to(lc_x[:, :, None, :], (B, G, K, C))], axis=-1)
    return _pose_geo(jnp.transpose(knn_xyz, (0, 3, 1, 2)),
                     jnp.transpose(knn_x, (0, 3, 1, 2)),
                     jnp.transpose(knn_rgb, (0, 3, 1, 2)), out_dim, vv)


def kernel(xyz, x, rgb, rgbx, vv):
    x = _pose_initial(x, rgbx)
    xyz_list = [xyz]
    x_list = [x]
    out_dim = _EMBED
    group_num = xyz.shape[1]
    for i in range(_STAGES):
        out_dim = out_dim * 2
        group_num = group_num // 2
        x_t = jnp.transpose(x, (0, 2, 1))
        fps_idx = _fps(xyz, group_num)
        lc_xyz = _index_points(xyz, fps_idx)
        lc_x = _index_points(x_t, fps_idx)
        lc_rgb = _index_points(rgb, fps_idx)
        knn_idx = _knn(_K, xyz, lc_xyz)
        knn_xyz = _index_points(xyz, knn_idx)
        knn_x = _index_points(x_t, knn_idx)
        knn_rgb = _index_points(rgb, knn_idx)
        knn_x_w = _lga(lc_xyz, lc_x, lc_rgb, knn_xyz, knn_x, knn_rgb, out_dim, vv)
        pooled = jnp.max(knn_x_w, axis=-1)
        x = jnp.where(pooled > 0, pooled, 0.1 * pooled)
        xyz = lc_xyz
        rgb = lc_rgb
        xyz_list.append(xyz)
        x_list.append(x)
    return (tuple(xyz_list), tuple(x_list))
